# baseline (device time: 91746 ns/iter reference)
import jax
import jax.numpy as jnp
from jax import lax
from jax.experimental import pallas as pl
from jax.experimental.pallas import tpu as pltpu

U = 32
RU = 128
NE = 16


def kernel(x):
    m, n = x.shape
    assert m == U * RU

    def body(x_hbm, out_ref, x_f32, own16, m_buf,
             lsem, msem, ysend, xsend):
        my_x = lax.axis_index("x")
        my_y = lax.axis_index("y")
        my_z = lax.axis_index("z")
        y_nbr = (my_x, 1 - my_y, my_z)
        x_nbr = (1 - my_x, my_y, my_z)

        def unit(ref, u):
            return ref.at[pl.ds(u * RU, RU)]

        load_order = []
        for k in range(NE):
            load_order.append(k)
            load_order.append(NE + k)
        lcopies = [None] * U
        for u in load_order:
            cp = pltpu.make_async_copy(
                unit(x_hbm, u), unit(x_f32, u), lsem.at[u]
            )
            cp.start()
            lcopies[u] = cp

        def send_unit(src_ref, u, dst_dev, sem):
            rd = pltpu.make_async_remote_copy(
                src_ref=unit(src_ref, u),
                dst_ref=unit(m_buf, u),
                send_sem=sem,
                recv_sem=msem.at[u],
                device_id=dst_dev,
                device_id_type=pl.DeviceIdType.MESH,
            )
            rd.start()
            return rd

        def wait_unit(u):
            rd = pltpu.make_async_remote_copy(
                src_ref=unit(m_buf, u),
                dst_ref=unit(m_buf, u),
                send_sem=msem.at[u],
                recv_sem=msem.at[u],
                device_id=y_nbr,
                device_id_type=pl.DeviceIdType.MESH,
            )
            rd.wait_recv()

        def cast_unit(u):
            lcopies[u].wait()
            own16[pl.ds(u * RU, RU), :] = (
                x_f32[pl.ds(u * RU, RU), :].astype(jnp.bfloat16)
            )

        def add_unit(u):
            out_ref[pl.ds(u * RU, RU), :] = (
                own16[pl.ds(u * RU, RU), :] + m_buf[pl.ds(u * RU, RU), :]
            )

        a_base = my_x * NE
        b_base = (1 - my_x) * NE

        barrier_sem = pltpu.get_barrier_semaphore()
        for dev in (y_nbr, x_nbr):
            pl.semaphore_signal(
                barrier_sem, inc=1,
                device_id=dev, device_id_type=pl.DeviceIdType.MESH,
            )
        pl.semaphore_wait(barrier_sem, 2)

        sends = []
        for k in range(NE):
            cast_unit(k)
            cast_unit(NE + k)
            sends.append(send_unit(own16, a_base + k, y_nbr, ysend.at[k]))
        for k in range(NE):
            ua = a_base + k
            ub = b_base + k
            wait_unit(ua)
            sends.append(send_unit(m_buf, ua, x_nbr, xsend.at[k]))
            wait_unit(ub)
            add_unit(ua)
            add_unit(ub)
        for rd in sends:
            rd.wait_send()

    return pl.pallas_call(
        body,
        out_shape=jax.ShapeDtypeStruct((m, n), jnp.bfloat16),
        in_specs=[pl.BlockSpec(memory_space=pl.ANY)],
        out_specs=pl.BlockSpec(memory_space=pltpu.VMEM),
        scratch_shapes=[
            pltpu.VMEM((m, n), jnp.float32),
            pltpu.VMEM((m, n), jnp.bfloat16),
            pltpu.VMEM((m, n), jnp.bfloat16),
            pltpu.SemaphoreType.DMA((U,)),
            pltpu.SemaphoreType.DMA((U,)),
            pltpu.SemaphoreType.DMA((NE,)),
            pltpu.SemaphoreType.DMA((NE,)),
        ],
        compiler_params=pltpu.CompilerParams(
            collective_id=0, vmem_limit_bytes=64 * 1024 * 1024
        ),
    )(x)


# device time: 73201 ns/iter; 1.2533x vs baseline; 1.2533x over previous
import jax
import jax.numpy as jnp
from jax import lax
from jax.experimental import pallas as pl
from jax.experimental.pallas import tpu as pltpu

U = 32
RU = 128
NE = 11
NB = 5
B_BASE = 2 * NE


def kernel(x):
    m, n = x.shape
    assert m == U * RU

    def body(x_hbm, out_ref, x_f32, own16, m_buf,
             lsem, msem, ysend, xsend, zasend, zbsend):
        my_x = lax.axis_index("x")
        my_y = lax.axis_index("y")
        my_z = lax.axis_index("z")

        def unit(ref, u):
            return ref.at[pl.ds(u * RU, RU)]

        load_order = []
        for k in range(NE):
            load_order.append(k)
            load_order.append(NE + k)
            if k < NB:
                load_order.append(B_BASE + k)
                load_order.append(B_BASE + NB + k)
        lcopies = [None] * U
        for u in load_order:
            cp = pltpu.make_async_copy(
                unit(x_hbm, u), unit(x_f32, u), lsem.at[u]
            )
            cp.start()
            lcopies[u] = cp

        barrier_sem = pltpu.get_barrier_semaphore()

        def send_unit(src_ref, u, dst_dev, sem):
            rd = pltpu.make_async_remote_copy(
                src_ref=unit(src_ref, u),
                dst_ref=unit(m_buf, u),
                send_sem=sem,
                recv_sem=msem.at[u],
                device_id=dst_dev,
                device_id_type=pl.DeviceIdType.MESH,
            )
            rd.start()
            return rd

        def wait_unit(u, dummy_dev):
            rd = pltpu.make_async_remote_copy(
                src_ref=unit(m_buf, u),
                dst_ref=unit(m_buf, u),
                send_sem=msem.at[u],
                recv_sem=msem.at[u],
                device_id=dummy_dev,
                device_id_type=pl.DeviceIdType.MESH,
            )
            rd.wait_recv()

        def cast_unit(u):
            lcopies[u].wait()
            own16[pl.ds(u * RU, RU), :] = (
                x_f32[pl.ds(u * RU, RU), :].astype(jnp.bfloat16)
            )

        def add_unit(u):
            out_ref[pl.ds(u * RU, RU), :] = (
                own16[pl.ds(u * RU, RU), :] + m_buf[pl.ds(u * RU, RU), :]
            )

        def cast_schedule(k):
            cast_unit(k)
            cast_unit(NE + k)
            if k < NB:
                cast_unit(B_BASE + k)
                cast_unit(B_BASE + NB + k)

        def emit_end(px, pz):
            zn = 1 if pz == 0 else 2
            push_col = 0 if pz == 0 else 1
            y_nbr = (px, 1 - my_y, pz)
            x_nbr = (1 - px, my_y, pz)
            z_dev = (px, my_y, zn)
            a_base = px * NE
            b_base = (1 - px) * NE
            p_base = push_col * NE

            for dev in (y_nbr, x_nbr, z_dev):
                pl.semaphore_signal(
                    barrier_sem, inc=1,
                    device_id=dev, device_id_type=pl.DeviceIdType.MESH,
                )
            pl.semaphore_wait(barrier_sem, 3)

            sends = []
            for k in range(NE):
                cast_schedule(k)
                sends.append(
                    send_unit(own16, a_base + k, y_nbr, ysend.at[k])
                )
            for k in range(NE):
                ua = a_base + k
                ub = b_base + k
                wait_unit(ua, y_nbr)
                sends.append(send_unit(m_buf, ua, x_nbr, xsend.at[k]))
                wait_unit(ub, y_nbr)
                sends.append(send_unit(m_buf, p_base + k, z_dev,
                                       zasend.at[k]))
                add_unit(ua)
                add_unit(ub)
            for k in range(2 * NB):
                ub = B_BASE + k
                wait_unit(ub, y_nbr)
                add_unit(ub)
            for rd in sends:
                rd.wait_send()

        def emit_mid(px, pz):
            end_z = 0 if pz == 1 else 3
            mid_z = 2 if pz == 1 else 1
            y_nbr = (px, 1 - my_y, pz)
            x_nbr = (1 - px, my_y, pz)
            end_dev = (px, my_y, end_z)
            mid_dev = (px, my_y, mid_z)
            a_base = B_BASE + px * NB
            b_base = B_BASE + (1 - px) * NB
            f_base = 0 if pz == 1 else NE
            t_base = NE if pz == 1 else 0

            for dev in (y_nbr, x_nbr, end_dev, mid_dev):
                pl.semaphore_signal(
                    barrier_sem, inc=1,
                    device_id=dev, device_id_type=pl.DeviceIdType.MESH,
                )
            pl.semaphore_wait(barrier_sem, 4)

            sends = []
            for k in range(NE):
                cast_schedule(k)
                if k < NB:
                    sends.append(
                        send_unit(own16, a_base + k, y_nbr, ysend.at[k])
                    )
            for k in range(NB):
                ua = a_base + k
                ub = b_base + k
                wait_unit(ua, y_nbr)
                sends.append(send_unit(m_buf, ua, x_nbr, xsend.at[k]))
                sends.append(send_unit(m_buf, ua, end_dev,
                                       zasend.at[2 * k]))
                wait_unit(ub, y_nbr)
                sends.append(send_unit(m_buf, ub, end_dev,
                                       zasend.at[2 * k + 1]))
                add_unit(ua)
                add_unit(ub)
            for k in range(NE):
                uf = f_base + k
                wait_unit(uf, y_nbr)
                sends.append(send_unit(m_buf, uf, mid_dev, zbsend.at[k]))
                add_unit(uf)
            for k in range(NE):
                ut = t_base + k
                wait_unit(ut, y_nbr)
                add_unit(ut)
            for rd in sends:
                rd.wait_send()

        for pz in range(4):
            for px in range(2):
                emit = emit_end if pz in (0, 3) else emit_mid

                @pl.when((my_z == pz) & (my_x == px))
                def _branch(emit=emit, px=px, pz=pz):
                    emit(px, pz)

    return pl.pallas_call(
        body,
        out_shape=jax.ShapeDtypeStruct((m, n), jnp.bfloat16),
        in_specs=[pl.BlockSpec(memory_space=pl.ANY)],
        out_specs=pl.BlockSpec(memory_space=pltpu.VMEM),
        scratch_shapes=[
            pltpu.VMEM((m, n), jnp.float32),
            pltpu.VMEM((m, n), jnp.bfloat16),
            pltpu.VMEM((m, n), jnp.bfloat16),
            pltpu.SemaphoreType.DMA((U,)),
            pltpu.SemaphoreType.DMA((U,)),
            pltpu.SemaphoreType.DMA((NE,)),
            pltpu.SemaphoreType.DMA((NE,)),
            pltpu.SemaphoreType.DMA((NE,)),
            pltpu.SemaphoreType.DMA((NE,)),
        ],
        compiler_params=pltpu.CompilerParams(
            collective_id=0, vmem_limit_bytes=64 * 1024 * 1024
        ),
    )(x)


# device time: 55339 ns/iter; 1.6579x vs baseline; 1.3228x over previous
import jax
import jax.numpy as jnp
from jax import lax
from jax.experimental import pallas as pl
from jax.experimental.pallas import tpu as pltpu

U = 32
RU = 128
NE = 11
NB = 5
B_BASE = 2 * NE


def kernel(x):
    m, n = x.shape
    assert m == U * RU

    def body(x_hbm, out_ref, x_f32, own16, m_buf,
             lsem, msem, ysend, xsend, zasend, zbsend):
        my_x = lax.axis_index("x")
        my_y = lax.axis_index("y")
        my_z = lax.axis_index("z")

        def unit(ref, u):
            return ref.at[pl.ds(u * RU, RU)]

        load_order = []
        for k in range(NE):
            load_order.append(k)
            load_order.append(NE + k)
            if k < NB:
                load_order.append(B_BASE + k)
                load_order.append(B_BASE + NB + k)
        lcopies = [None] * U
        for u in load_order:
            cp = pltpu.make_async_copy(
                unit(x_hbm, u), unit(x_f32, u), lsem.at[u]
            )
            cp.start()
            lcopies[u] = cp

        barrier_sem = pltpu.get_barrier_semaphore()

        def send_unit(src_ref, u, dst_dev, sem):
            rd = pltpu.make_async_remote_copy(
                src_ref=unit(src_ref, u),
                dst_ref=unit(m_buf, u),
                send_sem=sem,
                recv_sem=msem.at[u],
                device_id=dst_dev,
                device_id_type=pl.DeviceIdType.MESH,
            )
            rd.start()
            return rd

        def wait_unit(u, dummy_dev):
            rd = pltpu.make_async_remote_copy(
                src_ref=unit(m_buf, u),
                dst_ref=unit(m_buf, u),
                send_sem=msem.at[u],
                recv_sem=msem.at[u],
                device_id=dummy_dev,
                device_id_type=pl.DeviceIdType.MESH,
            )
            rd.wait_recv()

        def cast_unit(u):
            lcopies[u].wait()
            own16[pl.ds(u * RU, RU), :] = (
                x_f32[pl.ds(u * RU, RU), :].astype(jnp.bfloat16)
            )

        def add_unit(u):
            out_ref[pl.ds(u * RU, RU), :] = (
                own16[pl.ds(u * RU, RU), :] + m_buf[pl.ds(u * RU, RU), :]
            )

        def cast_schedule(k):
            cast_unit(k)
            cast_unit(NE + k)
            if k < NB:
                cast_unit(B_BASE + k)
                cast_unit(B_BASE + NB + k)

        def emit_end(px, pz):
            zn = 1 if pz == 0 else 2
            push_col = 0 if pz == 0 else 1
            y_nbr = (px, 1 - my_y, pz)
            x_nbr = (1 - px, my_y, pz)
            z_dev = (px, my_y, zn)
            a_base = px * NE
            b_base = (1 - px) * NE
            p_base = push_col * NE

            for dev in (y_nbr, x_nbr, z_dev):
                pl.semaphore_signal(
                    barrier_sem, inc=1,
                    device_id=dev, device_id_type=pl.DeviceIdType.MESH,
                )
            pl.semaphore_wait(barrier_sem, 3)

            push_mine = (push_col == px)

            sends = []

            def process_b(j):
                ub = b_base + j
                wait_unit(ub, y_nbr)
                if not push_mine:
                    sends.append(send_unit(m_buf, ub, z_dev,
                                           zasend.at[j]))
                add_unit(ub)

            for k in range(NE):
                cast_schedule(k)
                sends.append(
                    send_unit(own16, a_base + k, y_nbr, ysend.at[k])
                )
            for k in range(NE):
                ua = a_base + k
                wait_unit(ua, y_nbr)
                sends.append(send_unit(m_buf, ua, x_nbr, xsend.at[k]))
                if push_mine:
                    sends.append(send_unit(m_buf, ua, z_dev,
                                           zasend.at[k]))
                add_unit(ua)
                if k >= 2:
                    process_b(k - 2)
            for j in (NE - 2, NE - 1):
                process_b(j)
            for k in range(2 * NB):
                ub = B_BASE + k
                wait_unit(ub, y_nbr)
                add_unit(ub)
            for rd in sends:
                rd.wait_send()

        def emit_mid(px, pz):
            end_z = 0 if pz == 1 else 3
            mid_z = 2 if pz == 1 else 1
            y_nbr = (px, 1 - my_y, pz)
            x_nbr = (1 - px, my_y, pz)
            end_dev = (px, my_y, end_z)
            mid_dev = (px, my_y, mid_z)
            a_base = B_BASE + px * NB
            b_base = B_BASE + (1 - px) * NB
            f_base = 0 if pz == 1 else NE
            t_base = NE if pz == 1 else 0

            for dev in (y_nbr, x_nbr, end_dev, mid_dev):
                pl.semaphore_signal(
                    barrier_sem, inc=1,
                    device_id=dev, device_id_type=pl.DeviceIdType.MESH,
                )
            pl.semaphore_wait(barrier_sem, 4)

            sends = []

            def process_b(j):
                ub = b_base + j
                wait_unit(ub, y_nbr)
                sends.append(send_unit(m_buf, ub, end_dev,
                                       zasend.at[NB + j]))
                add_unit(ub)

            def process_t(j):
                ut = t_base + j
                wait_unit(ut, y_nbr)
                add_unit(ut)

            for k in range(NE):
                cast_schedule(k)
                if k < NB:
                    sends.append(
                        send_unit(own16, a_base + k, y_nbr, ysend.at[k])
                    )
            for k in range(NB):
                ua = a_base + k
                wait_unit(ua, y_nbr)
                sends.append(send_unit(m_buf, ua, x_nbr, xsend.at[k]))
                sends.append(send_unit(m_buf, ua, end_dev,
                                       zasend.at[k]))
                add_unit(ua)
                if k >= 2:
                    process_b(k - 2)
            for j in (NB - 2, NB - 1):
                process_b(j)
            for k in range(NE):
                uf = f_base + k
                wait_unit(uf, y_nbr)
                sends.append(send_unit(m_buf, uf, mid_dev, zbsend.at[k]))
                add_unit(uf)
                if k >= 2:
                    process_t(k - 2)
            for j in (NE - 2, NE - 1):
                process_t(j)
            for rd in sends:
                rd.wait_send()

        for pz in range(4):
            for px in range(2):
                emit = emit_end if pz in (0, 3) else emit_mid

                @pl.when((my_z == pz) & (my_x == px))
                def _branch(emit=emit, px=px, pz=pz):
                    emit(px, pz)

    return pl.pallas_call(
        body,
        out_shape=jax.ShapeDtypeStruct((m, n), jnp.bfloat16),
        in_specs=[pl.BlockSpec(memory_space=pl.ANY)],
        out_specs=pl.BlockSpec(memory_space=pltpu.VMEM),
        scratch_shapes=[
            pltpu.VMEM((m, n), jnp.float32),
            pltpu.VMEM((m, n), jnp.bfloat16),
            pltpu.VMEM((m, n), jnp.bfloat16),
            pltpu.SemaphoreType.DMA((U,)),
            pltpu.SemaphoreType.DMA((U,)),
            pltpu.SemaphoreType.DMA((NE,)),
            pltpu.SemaphoreType.DMA((NE,)),
            pltpu.SemaphoreType.DMA((NE,)),
            pltpu.SemaphoreType.DMA((NE,)),
        ],
        compiler_params=pltpu.CompilerParams(
            collective_id=0, vmem_limit_bytes=64 * 1024 * 1024
        ),
    )(x)
